# baseline (speedup 1.0000x reference)
import jax
import jax.numpy as jnp
from jax.experimental import pallas as pl
from jax.experimental.pallas import tpu as pltpu

B, T, D_IN, H = 16, 2048, 512, 64
CT = 512
NT = T // CT


def _fused_kernel(xt_ref, ml_ref, len_ref, Wf_ref, Wr_ref, Wih_ref, Whh_ref,
                  bf_ref, br_ref, bih_ref, bhh_ref, Wcls_ref, bcls_ref,
                  out_ref, Wbig_s, bbig_s, h_s, acc_s, A_s):
    i = pl.program_id(0)

    @pl.when(i == 0)
    def _init():
        Wbig = jnp.dot(jnp.dot(Wf_ref[...], Wr_ref[...],
                               preferred_element_type=jnp.float32),
                       Wih_ref[...], preferred_element_type=jnp.float32)
        bbig = (jnp.dot(jnp.dot(bf_ref[...], Wr_ref[...],
                                preferred_element_type=jnp.float32)
                        + br_ref[...], Wih_ref[...],
                        preferred_element_type=jnp.float32)
                + bih_ref[...] + bhh_ref[...])
        Wbig_s[...] = Wbig
        bbig_s[...] = bbig
        h_s[...] = jnp.zeros((B, H), jnp.float32)
        acc_s[...] = jnp.zeros((B, H), jnp.float32)

    t0 = i * CT
    maxlen = ml_ref[0]

    @pl.when(t0 < maxlen)
    def _gemm():
        xb = xt_ref[...].reshape(B * CT, D_IN)
        A = jnp.dot(xb, Wbig_s[...],
                    preferred_element_type=jnp.float32) + bbig_s[...]
        A_s[...] = A.reshape(B, CT, H).transpose(1, 0, 2).reshape(CT * B, H)

    h0 = h_s[...]
    acc0 = acc_s[...]
    Whh = Whh_ref[...]
    lens = len_ref[...]

    HB = B // 2

    def step(t, carry):
        h1, h2, acc1, acc2 = carry
        a = A_s[pl.ds(t * B, B), :]
        a1 = a[:HB, :]
        a2 = a[HB:, :]
        hn1 = jnp.tanh(a1 + jnp.dot(h1, Whh,
                                    preferred_element_type=jnp.float32))
        hn2 = jnp.tanh(a2 + jnp.dot(h2, Whh,
                                    preferred_element_type=jnp.float32))
        tt = t0 + t
        acc1 = acc1 + jnp.where(tt < lens[:HB, :], hn1, 0.0)
        acc2 = acc2 + jnp.where(tt < lens[HB:, :], hn2, 0.0)
        return (hn1, hn2, acc1, acc2)

    nsteps = jnp.clip(maxlen - t0, 0, CT)
    UN = 8
    n8 = nsteps // UN

    def outer(o, carry):
        for j in range(UN):
            carry = step(o * UN + j, carry)
        return carry

    def rem(r, carry):
        return step(n8 * UN + r, carry)

    carry0 = (h0[:HB, :], h0[HB:, :], acc0[:HB, :], acc0[HB:, :])
    carry = jax.lax.fori_loop(0, n8, outer, carry0)
    h1, h2, acc1, acc2 = jax.lax.fori_loop(0, nsteps - n8 * UN, rem, carry)
    h_s[...] = jnp.concatenate([h1, h2], axis=0)
    acc = jnp.concatenate([acc1, acc2], axis=0)
    acc_s[...] = acc

    @pl.when(i == NT - 1)
    def _finish():
        pooled = acc / lens.astype(jnp.float32)
        out_ref[...] = jnp.dot(pooled, Wcls_ref[...],
                               preferred_element_type=jnp.float32) + bcls_ref[...]


def kernel(x, lengths, W_feat, b_feat, W_red, b_red, W_ih, W_hh, b_ih, b_hh,
           W_cls, b_cls):
    lens2 = lengths.reshape(B, 1).astype(jnp.int32)
    ml = lengths[:1].astype(jnp.int32)
    bf = b_feat.reshape(1, -1)
    br = b_red.reshape(1, -1)
    bih = b_ih.reshape(1, -1)
    bhh = b_hh.reshape(1, -1)
    bcls = b_cls.reshape(1, -1)

    full = lambda shape: pl.BlockSpec(shape, lambda i: (0,) * len(shape))
    out = pl.pallas_call(
        _fused_kernel,
        grid=(NT,),
        in_specs=[
            pl.BlockSpec((B, CT, D_IN), lambda i: (0, i, 0)),
            pl.BlockSpec(memory_space=pltpu.SMEM),
            full((B, 1)),
            full((D_IN, D_IN)),
            full((D_IN, H)),
            full((H, H)),
            full((H, H)),
            full((1, D_IN)),
            full((1, H)),
            full((1, H)),
            full((1, H)),
            full((H, H)),
            full((1, H)),
        ],
        out_specs=full((B, H)),
        out_shape=jax.ShapeDtypeStruct((B, H), jnp.float32),
        scratch_shapes=[
            pltpu.VMEM((D_IN, H), jnp.float32),
            pltpu.VMEM((1, H), jnp.float32),
            pltpu.VMEM((B, H), jnp.float32),
            pltpu.VMEM((B, H), jnp.float32),
            pltpu.VMEM((CT * B, H), jnp.float32),
        ],
    )(x, ml, lens2, W_feat, W_red, W_ih, W_hh, bf, br, bih, bhh, W_cls, bcls)
    return out


# speedup vs baseline: 1.0059x; 1.0059x over previous
import jax
import jax.numpy as jnp
from jax.experimental import pallas as pl
from jax.experimental.pallas import tpu as pltpu

B, T, D_IN, H = 16, 2048, 512, 64
CT = 256
NT = T // CT


def _fused_kernel(xt_ref, ml_ref, len_ref, Wf_ref, Wr_ref, Wih_ref, Whh_ref,
                  bf_ref, br_ref, bih_ref, bhh_ref, Wcls_ref, bcls_ref,
                  out_ref, Wbig_s, bbig_s, h_s, acc_s, A_s):
    i = pl.program_id(0)

    @pl.when(i == 0)
    def _init():
        Wbig = jnp.dot(jnp.dot(Wf_ref[...], Wr_ref[...],
                               preferred_element_type=jnp.float32),
                       Wih_ref[...], preferred_element_type=jnp.float32)
        bbig = (jnp.dot(jnp.dot(bf_ref[...], Wr_ref[...],
                                preferred_element_type=jnp.float32)
                        + br_ref[...], Wih_ref[...],
                        preferred_element_type=jnp.float32)
                + bih_ref[...] + bhh_ref[...])
        Wbig_s[...] = Wbig
        bbig_s[...] = bbig
        h_s[...] = jnp.zeros((B, H), jnp.float32)
        acc_s[...] = jnp.zeros((B, H), jnp.float32)

    t0 = i * CT
    maxlen = ml_ref[0]

    @pl.when(t0 < maxlen)
    def _gemm():
        xb = xt_ref[...].reshape(B * CT, D_IN)
        A = jnp.dot(xb, Wbig_s[...],
                    preferred_element_type=jnp.float32) + bbig_s[...]
        A_s[...] = A.reshape(B, CT, H).transpose(1, 0, 2).reshape(CT * B, H)

    h0 = h_s[...]
    acc0 = acc_s[...]
    Whh = Whh_ref[...]
    lens = len_ref[...]

    HB = B // 2

    def step(t, carry):
        h1, h2, acc1, acc2 = carry
        a = A_s[pl.ds(t * B, B), :]
        a1 = a[:HB, :]
        a2 = a[HB:, :]
        hn1 = jnp.tanh(a1 + jnp.dot(h1, Whh,
                                    preferred_element_type=jnp.float32))
        hn2 = jnp.tanh(a2 + jnp.dot(h2, Whh,
                                    preferred_element_type=jnp.float32))
        tt = t0 + t
        acc1 = acc1 + jnp.where(tt < lens[:HB, :], hn1, 0.0)
        acc2 = acc2 + jnp.where(tt < lens[HB:, :], hn2, 0.0)
        return (hn1, hn2, acc1, acc2)

    nsteps = jnp.clip(maxlen - t0, 0, CT)
    UN = 8
    n8 = nsteps // UN

    def outer(o, carry):
        for j in range(UN):
            carry = step(o * UN + j, carry)
        return carry

    def rem(r, carry):
        return step(n8 * UN + r, carry)

    carry0 = (h0[:HB, :], h0[HB:, :], acc0[:HB, :], acc0[HB:, :])
    carry = jax.lax.fori_loop(0, n8, outer, carry0)
    h1, h2, acc1, acc2 = jax.lax.fori_loop(0, nsteps - n8 * UN, rem, carry)
    h_s[...] = jnp.concatenate([h1, h2], axis=0)
    acc = jnp.concatenate([acc1, acc2], axis=0)
    acc_s[...] = acc

    @pl.when(i == NT - 1)
    def _finish():
        pooled = acc / lens.astype(jnp.float32)
        out_ref[...] = jnp.dot(pooled, Wcls_ref[...],
                               preferred_element_type=jnp.float32) + bcls_ref[...]


def kernel(x, lengths, W_feat, b_feat, W_red, b_red, W_ih, W_hh, b_ih, b_hh,
           W_cls, b_cls):
    lens2 = lengths.reshape(B, 1).astype(jnp.int32)
    ml = lengths[:1].astype(jnp.int32)
    bf = b_feat.reshape(1, -1)
    br = b_red.reshape(1, -1)
    bih = b_ih.reshape(1, -1)
    bhh = b_hh.reshape(1, -1)
    bcls = b_cls.reshape(1, -1)

    full = lambda shape: pl.BlockSpec(shape, lambda i: (0,) * len(shape))
    out = pl.pallas_call(
        _fused_kernel,
        grid=(NT,),
        in_specs=[
            pl.BlockSpec((B, CT, D_IN), lambda i: (0, i, 0)),
            pl.BlockSpec(memory_space=pltpu.SMEM),
            full((B, 1)),
            full((D_IN, D_IN)),
            full((D_IN, H)),
            full((H, H)),
            full((H, H)),
            full((1, D_IN)),
            full((1, H)),
            full((1, H)),
            full((1, H)),
            full((H, H)),
            full((1, H)),
        ],
        out_specs=full((B, H)),
        out_shape=jax.ShapeDtypeStruct((B, H), jnp.float32),
        scratch_shapes=[
            pltpu.VMEM((D_IN, H), jnp.float32),
            pltpu.VMEM((1, H), jnp.float32),
            pltpu.VMEM((B, H), jnp.float32),
            pltpu.VMEM((B, H), jnp.float32),
            pltpu.VMEM((CT * B, H), jnp.float32),
        ],
    )(x, ml, lens2, W_feat, W_red, W_ih, W_hh, bf, br, bih, bhh, W_cls, bcls)
    return out
